# baseline (speedup 1.0000x reference)
import jax
import jax.numpy as jnp
from jax.experimental import pallas as pl
from jax.experimental.pallas import tpu as pltpu

BATCH = 4096
N_CLUSTERS = 8192
R = 256
CK = 256


def _assign_kernel(x_ref, c_ref, out_ref):
    x0 = x_ref[:, 0:1]
    x1 = x_ref[:, 1:2]

    def body(t, carry):
        bestv, bidx = carry
        c0 = c_ref[0:1, pl.ds(t * CK, CK)]
        c1 = c_ref[1:2, pl.ds(t * CK, CK)]
        d0 = x0 - c0
        d1 = x1 - c1
        dist = d0 * d0 + d1 * d1
        mask = dist < bestv
        bestv = jnp.where(mask, dist, bestv)
        bidx = jnp.where(mask, t, bidx)
        return bestv, bidx

    bestv0 = jnp.full((R, CK), jnp.inf, dtype=jnp.float32)
    bidx0 = jnp.zeros((R, CK), dtype=jnp.int32)
    bestv, bidx = jax.lax.fori_loop(0, N_CLUSTERS // CK, body, (bestv0, bidx0))

    m = jnp.min(bestv, axis=-1, keepdims=True)
    lane = jax.lax.broadcasted_iota(jnp.int32, (R, CK), 1)
    cand = jnp.where(bestv == m, bidx * CK + lane, N_CLUSTERS)
    out_ref[:] = jnp.min(cand, axis=-1)


def kernel(inputs, cluster_centers):
    centers_t = cluster_centers.T
    grid = (BATCH // R,)
    return pl.pallas_call(
        _assign_kernel,
        grid=grid,
        in_specs=[
            pl.BlockSpec((R, 2), lambda i: (i, 0)),
            pl.BlockSpec((2, N_CLUSTERS), lambda i: (0, 0)),
        ],
        out_specs=pl.BlockSpec((R,), lambda i: (i,)),
        out_shape=jax.ShapeDtypeStruct((BATCH,), jnp.int32),
        compiler_params=pltpu.CompilerParams(
            dimension_semantics=("parallel",),
        ),
    )(inputs, centers_t)
